# Initial kernel scaffold; baseline (speedup 1.0000x reference)
import jax
import jax.numpy as jnp
from jax.experimental import pallas as pl


def kernel(x, edge_index, edge_attr, batch_idx, W_l1, b_l1, W_r1, b_r1, W_e1, att1, bias1, gn_w, gn_b, gn_ms, W_l2, b_l2, W_r2, b_r2, W_e2, att2, bias2):
    raise NotImplementedError("write your pallas kernel here")


# speedup vs baseline: 12.3080x; 12.3080x over previous
import functools

import numpy as np
import jax
import jax.numpy as jnp
from jax import lax
from jax.experimental import pallas as pl
from jax.experimental.pallas import tpu as pltpu
from jax.experimental.pallas import tpu_sc as plsc

N = 10000
E = 320000
NB = 64
H = 10

NC = 2
NS = 16
NW = NC * NS
CH = 80
NPAD = 10240


def _sc_gather(num_tables, table_rows, B, D):
    per_w = B // NW
    nch = per_w // CH
    mesh = plsc.VectorSubcoreMesh(core_axis_name="c", subcore_axis_name="s")
    out_type = tuple(jax.ShapeDtypeStruct((B, D), jnp.float32)
                     for _ in range(num_tables))
    scratch = []
    for _ in range(num_tables):
        scratch += [pltpu.VMEM((CH,), jnp.int32),
                    pltpu.VMEM((CH, D), jnp.float32),
                    pltpu.SemaphoreType.DMA]

    @functools.partial(pl.kernel, mesh=mesh, out_type=out_type,
                       scratch_types=scratch,
                       compiler_params=pltpu.CompilerParams(
                           use_tc_tiling_on_sc=False))
    def k(*refs):
        tables = refs[:num_tables]
        idxs = refs[num_tables:2 * num_tables]
        outs = refs[2 * num_tables:3 * num_tables]
        scr = refs[3 * num_tables:]
        wid = lax.axis_index("s") * NC + lax.axis_index("c")
        base = wid * per_w

        def body(j, carry):
            off = base + j * CH
            for t in range(num_tables):
                idx_v, rows_v, sem = scr[3 * t:3 * t + 3]
                pltpu.sync_copy(idxs[t].at[pl.ds(off, CH)], idx_v)
                pltpu.async_copy(tables[t].at[idx_v], rows_v, sem).wait()
                pltpu.sync_copy(rows_v, outs[t].at[pl.ds(off, CH)])
            return carry

        lax.fori_loop(0, nch, body, 0)

    return k


def _sc_scatter_add(B, D, NT):
    per_w = B // NW
    nch = per_w // CH
    rps = NT // NS
    mesh = plsc.VectorSubcoreMesh(core_axis_name="c", subcore_axis_name="s")

    @functools.partial(
        pl.kernel, mesh=mesh,
        out_type=jax.ShapeDtypeStruct((NC, NT, D), jnp.float32),
        scratch_types=[pltpu.VMEM_SHARED((NT, D), jnp.float32),
                       pltpu.VMEM((CH,), jnp.int32),
                       pltpu.VMEM((CH, D), jnp.float32)],
        compiler_params=pltpu.CompilerParams(use_tc_tiling_on_sc=False))
    def k(vals_hbm, idx_hbm, zeros_hbm, out_hbm, acc, idx_v, vals_v):
        c = lax.axis_index("c")
        s = lax.axis_index("s")
        pltpu.sync_copy(zeros_hbm, acc.at[pl.ds(s * rps, rps)])
        plsc.subcore_barrier()
        wid = s * NC + c
        base = wid * per_w

        def body(j, carry):
            off = base + j * CH
            pltpu.sync_copy(idx_hbm.at[pl.ds(off, CH)], idx_v)
            pltpu.sync_copy(vals_hbm.at[pl.ds(off, CH)], vals_v)
            pltpu.sync_copy(vals_v, acc.at[idx_v], add=True)
            return carry

        lax.fori_loop(0, nch, body, 0)
        plsc.subcore_barrier()
        pltpu.sync_copy(acc.at[pl.ds(s * rps, rps)],
                        out_hbm.at[c, pl.ds(s * rps, rps)])

    return k


NBLK = 1000
EBLK = 4000


def _full(shape):
    return pl.BlockSpec(shape, lambda i: tuple(0 for _ in shape))


def _tc_proj1(x, Wl, bl, Wr, br):
    def body(x_ref, wl_ref, bl_ref, wr_ref, br_ref, xl_ref, xr_ref):
        xb = x_ref[...]
        xl_ref[...] = jnp.dot(xb, wl_ref[...],
                              preferred_element_type=jnp.float32) + bl_ref[...]
        xr_ref[...] = jnp.dot(xb, wr_ref[...],
                              preferred_element_type=jnp.float32) + br_ref[...]

    return pl.pallas_call(
        body,
        grid=(N // NBLK,),
        in_specs=[pl.BlockSpec((NBLK, 128), lambda i: (i, 0)),
                  _full((128, 64)), _full((1, 64)),
                  _full((128, 64)), _full((1, 64))],
        out_specs=[pl.BlockSpec((NBLK, 64), lambda i: (i, 0))] * 2,
        out_shape=[jax.ShapeDtypeStruct((N, 64), jnp.float32)] * 2,
    )(x, Wl, bl, Wr, br)


def _tc_alpha1(xsrc, xdst, eattr, Wep, attb, G, pmask):
    def body(xs_ref, xd_ref, ea_ref, we_ref, att_ref, g_ref, pm_ref,
             alpha_ref, p_ref):
        z = xs_ref[...] + xd_ref[...] + jnp.dot(
            ea_ref[...], we_ref[...], preferred_element_type=jnp.float32)
        z = jnp.where(z >= 0, z, 0.2 * z)
        t = z * att_ref[...]
        alpha = jnp.dot(t, g_ref[...], preferred_element_type=jnp.float32)
        q = jnp.clip(jnp.floor(alpha + 0.5), -120.0, 100.0)
        alpha_ref[...] = alpha
        p_ref[...] = jnp.exp2(q) * pm_ref[...]

    return pl.pallas_call(
        body,
        grid=(E // EBLK,),
        in_specs=[pl.BlockSpec((EBLK, 64), lambda i: (i, 0)),
                  pl.BlockSpec((EBLK, 64), lambda i: (i, 0)),
                  pl.BlockSpec((EBLK, 16), lambda i: (i, 0)),
                  _full((16, 64)), _full((1, 64)), _full((64, 16)),
                  _full((1, 16))],
        out_specs=[pl.BlockSpec((EBLK, 16), lambda i: (i, 0))] * 2,
        out_shape=[jax.ShapeDtypeStruct((E, 16), jnp.float32)] * 2,
    )(xsrc, xdst, eattr, Wep, attb, G, pmask)


def _tc_mprime(Spow, NT):
    blk = min(NT, 2000)

    def body(s_ref, m_ref):
        S = s_ref[0] + s_ref[1]
        m_ref[...] = jnp.where(S > 0, jnp.floor(jnp.log2(
            jnp.maximum(S, 1e-38))), 0.0)

    return pl.pallas_call(
        body,
        grid=(NT // blk,),
        in_specs=[pl.BlockSpec((2, blk, 16), lambda i: (0, i, 0))],
        out_specs=pl.BlockSpec((blk, 16), lambda i: (i, 0)),
        out_shape=jax.ShapeDtypeStruct((NT, 16), jnp.float32),
    )(Spow)


def _tc_eaw1(alpha, mdst, xsrc, GT, pmask):
    def body(a_ref, m_ref, xs_ref, gt_ref, pm_ref, out_ref):
        ea = jnp.exp(a_ref[...] - m_ref[...]) * pm_ref[...]
        eaex = jnp.dot(ea, gt_ref[...], preferred_element_type=jnp.float32)
        w = xs_ref[...] * eaex
        out_ref[...] = jnp.concatenate([ea, w], axis=1)

    return pl.pallas_call(
        body,
        grid=(E // EBLK,),
        in_specs=[pl.BlockSpec((EBLK, 16), lambda i: (i, 0)),
                  pl.BlockSpec((EBLK, 16), lambda i: (i, 0)),
                  pl.BlockSpec((EBLK, 64), lambda i: (i, 0)),
                  _full((16, 64)), _full((1, 16))],
        out_specs=pl.BlockSpec((EBLK, 80), lambda i: (i, 0)),
        out_shape=jax.ShapeDtypeStruct((E, 80), jnp.float32),
    )(alpha, mdst, xsrc, GT, pmask)


def _tc_out1gn(acc, GT, Gm, bias1):
    def body(a_ref, gt_ref, gm_ref, b_ref, h1_ref, gn_ref):
        A = a_ref[0] + a_ref[1]
        S = A[:, :16]
        W = A[:, 16:]
        Sx = jnp.dot(S, gt_ref[...], preferred_element_type=jnp.float32)
        out = jnp.where(Sx > 0, W / jnp.where(Sx > 0, Sx, 1.0), 0.0)
        h1 = jnp.dot(out, gm_ref[...],
                     preferred_element_type=jnp.float32) + b_ref[...]
        h1_ref[...] = h1
        hc = h1[:, :5]
        gn_ref[...] = jnp.concatenate(
            [hc, hc * hc,
             jnp.ones((hc.shape[0], 1), jnp.float32),
             jnp.zeros((hc.shape[0], 5), jnp.float32)], axis=1)

    return pl.pallas_call(
        body,
        grid=(N // NBLK,),
        in_specs=[pl.BlockSpec((2, NBLK, 80), lambda i: (0, i, 0)),
                  _full((16, 64)), _full((64, 8)), _full((1, 8))],
        out_specs=[pl.BlockSpec((NBLK, 8), lambda i: (i, 0)),
                   pl.BlockSpec((NBLK, 16), lambda i: (i, 0))],
        out_shape=[jax.ShapeDtypeStruct((N, 8), jnp.float32),
                   jax.ShapeDtypeStruct((N, 16), jnp.float32)],
    )(acc, GT, Gm, bias1)


def _tc_gnstats(acc, gn_w, gn_ms):
    def body(a_ref, w_ref, ms_ref, g_ref):
        A = a_ref[0] + a_ref[1]
        cnt = jnp.maximum(A[:, 10:11], 1.0)
        mean = A[:, 0:5] / cnt
        Eh2 = A[:, 5:10] / cnt
        ms = ms_ref[...][:, :5]
        w = w_ref[...][:, :5]
        var = Eh2 - mean * mean * ms * (2.0 - ms)
        std = jnp.sqrt(var + 1e-5)
        g_ref[...] = jnp.concatenate(
            [mean * ms, w / std, jnp.zeros((A.shape[0], 6), jnp.float32)],
            axis=1)

    return pl.pallas_call(
        body,
        grid=(1,),
        in_specs=[_full((2, NB, 16)), _full((1, 8)), _full((1, 8))],
        out_specs=_full((NB, 16)),
        out_shape=jax.ShapeDtypeStruct((NB, 16), jnp.float32),
    )(acc, gn_w, gn_ms)


def _tc_normproj2(h1, gb, gnb, Wl2, bl2, Wr2, br2):
    def body(h_ref, g_ref, b_ref, wl_ref, bl_ref, wr_ref, br_ref,
             xl_ref, xr_ref):
        h = h_ref[...]
        g = g_ref[...]
        hn5 = (h[:, :5] - g[:, :5]) * g[:, 5:10] + b_ref[...][:, :5]
        hn5 = jnp.maximum(hn5, 0.0)
        hn8 = jnp.concatenate(
            [hn5, jnp.zeros((hn5.shape[0], 3), jnp.float32)], axis=1)
        xl_ref[...] = jnp.dot(hn8, wl_ref[...],
                              preferred_element_type=jnp.float32) + bl_ref[...]
        xr_ref[...] = jnp.dot(hn8, wr_ref[...],
                              preferred_element_type=jnp.float32) + br_ref[...]

    return pl.pallas_call(
        body,
        grid=(N // NBLK,),
        in_specs=[pl.BlockSpec((NBLK, 8), lambda i: (i, 0)),
                  pl.BlockSpec((NBLK, 16), lambda i: (i, 0)),
                  _full((1, 8)), _full((8, 16)), _full((1, 16)),
                  _full((8, 16)), _full((1, 16))],
        out_specs=[pl.BlockSpec((NBLK, 16), lambda i: (i, 0))] * 2,
        out_shape=[jax.ShapeDtypeStruct((N, 16), jnp.float32)] * 2,
    )(h1, gb, gnb, Wl2, bl2, Wr2, br2)


def _tc_alpha2(xsrc, xdst, eattr, Wep, attb, pmask):
    def body(xs_ref, xd_ref, ea_ref, we_ref, att_ref, pm_ref,
             alpha_ref, p_ref):
        z = xs_ref[...] + xd_ref[...] + jnp.dot(
            ea_ref[...], we_ref[...], preferred_element_type=jnp.float32)
        z = jnp.where(z >= 0, z, 0.2 * z)
        alpha = z * att_ref[...] * pm_ref[...]
        q = jnp.clip(jnp.floor(alpha + 0.5), -120.0, 100.0)
        alpha_ref[...] = alpha
        p_ref[...] = jnp.exp2(q) * pm_ref[...]

    return pl.pallas_call(
        body,
        grid=(E // EBLK,),
        in_specs=[pl.BlockSpec((EBLK, 16), lambda i: (i, 0)),
                  pl.BlockSpec((EBLK, 16), lambda i: (i, 0)),
                  pl.BlockSpec((EBLK, 16), lambda i: (i, 0)),
                  _full((16, 16)), _full((1, 16)), _full((1, 16))],
        out_specs=[pl.BlockSpec((EBLK, 16), lambda i: (i, 0))] * 2,
        out_shape=[jax.ShapeDtypeStruct((E, 16), jnp.float32)] * 2,
    )(xsrc, xdst, eattr, Wep, attb, pmask)


def _tc_eaw2(alpha, mdst, xsrc, pmask):
    def body(a_ref, m_ref, xs_ref, pm_ref, out_ref):
        ea = jnp.exp(a_ref[...] - m_ref[...]) * pm_ref[...]
        out_ref[...] = jnp.concatenate([ea, xs_ref[...] * ea], axis=1)

    return pl.pallas_call(
        body,
        grid=(E // EBLK,),
        in_specs=[pl.BlockSpec((EBLK, 16), lambda i: (i, 0)),
                  pl.BlockSpec((EBLK, 16), lambda i: (i, 0)),
                  pl.BlockSpec((EBLK, 16), lambda i: (i, 0)),
                  _full((1, 16))],
        out_specs=pl.BlockSpec((EBLK, 32), lambda i: (i, 0)),
        out_shape=jax.ShapeDtypeStruct((E, 32), jnp.float32),
    )(alpha, mdst, xsrc, pmask)


def _tc_final(acc, Mh, bias2):
    def body(a_ref, mh_ref, b_ref, out_ref):
        A = a_ref[0] + a_ref[1]
        S = A[:, :16]
        W = A[:, 16:]
        out = jnp.where(S > 0, W / jnp.where(S > 0, S, 1.0), 0.0)
        r = jnp.dot(out, mh_ref[...],
                    preferred_element_type=jnp.float32) + b_ref[...]
        out_ref[...] = jax.nn.sigmoid(r)

    return pl.pallas_call(
        body,
        grid=(N // NBLK,),
        in_specs=[pl.BlockSpec((2, NBLK, 32), lambda i: (0, i, 0)),
                  _full((16, 8)), _full((1, 8))],
        out_specs=pl.BlockSpec((NBLK, 8), lambda i: (i, 0)),
        out_shape=jax.ShapeDtypeStruct((N, 8), jnp.float32),
    )(acc, Mh, bias2)


_G = np.zeros((64, 16), np.float32)
for _h in range(H):
    _G[5 * _h:5 * _h + 5, _h] = 1.0
_GT = np.ascontiguousarray(_G.T)
_GM = np.zeros((64, 8), np.float32)
for _h in range(H):
    for _c in range(5):
        _GM[5 * _h + _c, _c] = 1.0 / H
_MH = np.zeros((16, 8), np.float32)
_MH[:H, 0] = 1.0 / H
_PMASK = np.zeros((1, 16), np.float32)
_PMASK[0, :H] = 1.0


def _pad2(a, r, c):
    a = jnp.asarray(a, jnp.float32)
    if a.ndim == 1:
        a = a[None, :]
    return jnp.pad(a, ((0, r - a.shape[0]), (0, c - a.shape[1])))


def kernel(x, edge_index, edge_attr, batch_idx, W_l1, b_l1, W_r1, b_r1,
           W_e1, att1, bias1, gn_w, gn_b, gn_ms, W_l2, b_l2, W_r2, b_r2,
           W_e2, att2, bias2):
    src = edge_index[0]
    dst = edge_index[1]

    G = jnp.asarray(_G)
    GT = jnp.asarray(_GT)
    Gm = jnp.asarray(_GM)
    Mh = jnp.asarray(_MH)
    pmask = jnp.asarray(_PMASK)

    Wl1p = _pad2(W_l1, 128, 64)
    bl1p = _pad2(b_l1, 1, 64)
    Wr1p = _pad2(W_r1, 128, 64)
    br1p = _pad2(b_r1, 1, 64)
    We1p = _pad2(W_e1, 16, 64)
    att1b = _pad2(att1.reshape(-1), 1, 64)
    bias1p = _pad2(bias1, 1, 8)
    gnwp = _pad2(gn_w, 1, 8)
    gnmsp = _pad2(gn_ms, 1, 8)
    gnbp = _pad2(gn_b, 1, 8)
    Wl2p = _pad2(W_l2, 8, 16)
    bl2p = _pad2(b_l2, 1, 16)
    Wr2p = _pad2(W_r2, 8, 16)
    br2p = _pad2(b_r2, 1, 16)
    We2p = _pad2(W_e2, 16, 16)
    att2b = _pad2(att2[:, 0], 1, 16)
    bias2p = _pad2(bias2, 1, 8)

    z625_80 = jnp.zeros((N // NS, 80), jnp.float32)
    z625_16 = jnp.zeros((N // NS, 16), jnp.float32)
    z625_32 = jnp.zeros((N // NS, 32), jnp.float32)
    z4_16 = jnp.zeros((NB // NS, 16), jnp.float32)

    batch_pad = jnp.pad(batch_idx, (0, NPAD - N))
    npad_idx = jnp.pad(jnp.arange(N, dtype=jnp.int32), (0, NPAD - N))

    xl1, xr1 = _tc_proj1(x, Wl1p, bl1p, Wr1p, br1p)
    xsrc1, xdst1 = _sc_gather(2, N, E, 64)(xl1, xr1, src, dst)
    alpha1, p1 = _tc_alpha1(xsrc1, xdst1, edge_attr, We1p, att1b, G, pmask)
    Spow1 = _sc_scatter_add(E, 16, N)(p1, dst, z625_16)
    m1 = _tc_mprime(Spow1, N)
    (mdst1,) = _sc_gather(1, N, E, 16)(m1, dst)
    eaw1 = _tc_eaw1(alpha1, mdst1, xsrc1, GT, pmask)
    acc1 = _sc_scatter_add(E, 80, N)(eaw1, dst, z625_80)
    h1, gnrows = _tc_out1gn(acc1, GT, Gm, bias1p)

    gnrows_pad = jnp.pad(gnrows, ((0, NPAD - N), (0, 0)))
    gacc = _sc_scatter_add(NPAD, 16, NB)(gnrows_pad, batch_pad, z4_16)
    g = _tc_gnstats(gacc, gnwp, gnmsp)
    (gb_pad,) = _sc_gather(1, NB, NPAD, 16)(g, batch_pad)
    gb = gb_pad[:N]

    xl2, xr2 = _tc_normproj2(h1, gb, gnbp, Wl2p, bl2p, Wr2p, br2p)
    xsrc2, xdst2 = _sc_gather(2, N, E, 16)(xl2, xr2, src, dst)
    alpha2, p2 = _tc_alpha2(xsrc2, xdst2, edge_attr, We2p, att2b, pmask)
    Spow2 = _sc_scatter_add(E, 16, N)(p2, dst, z625_16)
    m2 = _tc_mprime(Spow2, N)
    (mdst2,) = _sc_gather(1, N, E, 16)(m2, dst)
    eaw2 = _tc_eaw2(alpha2, mdst2, xsrc2, pmask)
    acc2 = _sc_scatter_add(E, 32, N)(eaw2, dst, z625_32)
    out = _tc_final(acc2, Mh, bias2p)

    return out[:, :1]
